# baseline (speedup 1.0000x reference)
import jax
import jax.numpy as jnp
from jax import lax
from jax.experimental import pallas as pl
from jax.experimental.pallas import tpu as pltpu

SEG = 8
NB = 8
F = NB * SEG
C = 512
FOLD = 128
H = 28
W = 28
G = 2
NCB = C // FOLD


def _sigmoid(v):
    return 1.0 / (1.0 + jnp.exp(-v))


def _body(fold_ref, pass_ref, ocw_ref, ocb_ref, f1w_ref, f1b_ref, f2w_ref,
          f2b_ref, wcw_ref, wcb_ref, out_ref, cache_ref, pool_ref, m_ref):
    p = pl.program_id(0)
    b = pl.program_id(1)
    lin = p * NB + b
    q = lin // 3

    @pl.when(p != 3)
    def _pass_phase():
        @pl.when(lin % 3 == 0)
        def _pool():
            blk = fold_ref[...]
            cache_ref[q] = blk
            pool_ref[q] = jnp.sum(blk, axis=(0, 1)) * (1.0 / (H * W))

        out_ref[...] = pass_ref[...]

    @pl.when(p == 3)
    def _blend_phase():
        @pl.when(b == 0)
        def _nets():
            pool = pool_ref[...]
            zero = jnp.zeros((NB, 1, FOLD), jnp.float32)
            p_m1 = jnp.concatenate([zero, pool[:, : SEG - 1]], axis=1)
            p_p1 = jnp.concatenate([pool[:, 1:], zero], axis=1)

            ocw = ocw_ref[...]
            t0 = (jnp.sum(p_m1 * ocw[0][None, None, :], axis=-1)
                  + jnp.sum(pool * ocw[1][None, None, :], axis=-1)
                  + jnp.sum(p_p1 * ocw[2][None, None, :], axis=-1)
                  + ocb_ref[0, 0])
            t1 = jnp.maximum(
                jnp.dot(t0, f1w_ref[...], preferred_element_type=jnp.float32)
                + f1b_ref[0][None, :], 0.0)
            t2 = (jnp.dot(t1, f2w_ref[...], preferred_element_type=jnp.float32)
                  + f2b_ref[0][None, :])
            x_offset = 4.0 * (_sigmoid(t2) - 0.5)
            off_bs = jnp.tile(x_offset, (1, SEG // G))

            off0f = jnp.floor(off_bs)
            o0 = jnp.clip(off0f.astype(jnp.int32), 0, SEG - 1)
            o1 = jnp.clip(off0f.astype(jnp.int32) + 1, 0, SEG - 1)
            w1 = off_bs - off0f
            w0 = 1.0 - w1

            wcw = wcw_ref[...]
            xw = []
            for g in range(G):
                ug = (jnp.sum(p_m1 * wcw[0, :, g][None, None, :], axis=-1)
                      + jnp.sum(pool * wcw[1, :, g][None, None, :], axis=-1)
                      + jnp.sum(p_p1 * wcw[2, :, g][None, None, :], axis=-1)
                      + wcb_ref[0, g])
                xw.append(2.0 * _sigmoid(ug))

            fi = lax.broadcasted_iota(jnp.int32, (NB, SEG, SEG), 2)
            m4 = (jnp.where(fi == o0[:, :, None], w0[:, :, None], 0.0)
                  + jnp.where(fi == o1[:, :, None], w1[:, :, None], 0.0))
            xw_chan = jnp.concatenate(
                [jnp.broadcast_to(xw[0][:, :, None], (NB, SEG, FOLD // G)),
                 jnp.broadcast_to(xw[1][:, :, None], (NB, SEG, FOLD // G))],
                axis=-1)
            m_ref[...] = m4[:, :, :, None] * xw_chan[:, :, None, :]

        m = m_ref[b]
        blk = cache_ref[b]
        acc = blk[:, :, 0, :][:, :, None, :] * m[:, 0, :][None, None, :, :]
        for f in range(1, SEG):
            acc += blk[:, :, f, :][:, :, None, :] * m[:, f, :][None, None, :, :]
        out_ref[...] = acc


def kernel(x, off_conv_w, off_conv_b, off_fc1_w, off_fc1_b, off_fc2_w,
           off_fc2_b, w_conv_w, w_conv_b):
    xt = jnp.transpose(x, (2, 3, 0, 1))
    blk = (H, W, SEG, FOLD)
    out = pl.pallas_call(
        _body,
        grid=(NCB, NB),
        in_specs=[
            pl.BlockSpec(blk, lambda p, b: (0, 0,
                                            jnp.minimum((p * NB + b) // 3,
                                                        NB - 1), 0)),
            pl.BlockSpec(blk, lambda p, b: (0, 0,
                                            jnp.where(p == 3, NB - 1, b),
                                            jnp.where(p == 3, 3, p + 1))),
            pl.BlockSpec((3, FOLD), lambda p, b: (0, 0)),
            pl.BlockSpec((1, 1), lambda p, b: (0, 0)),
            pl.BlockSpec((SEG, SEG), lambda p, b: (0, 0)),
            pl.BlockSpec((1, SEG), lambda p, b: (0, 0)),
            pl.BlockSpec((SEG, G), lambda p, b: (0, 0)),
            pl.BlockSpec((1, G), lambda p, b: (0, 0)),
            pl.BlockSpec((3, FOLD, G), lambda p, b: (0, 0, 0)),
            pl.BlockSpec((1, G), lambda p, b: (0, 0)),
        ],
        out_specs=pl.BlockSpec(blk, lambda p, b: (0, 0, b,
                                                  jnp.where(p == 3, 0, p + 1))),
        out_shape=jax.ShapeDtypeStruct((H, W, F, C), jnp.float32),
        scratch_shapes=[
            pltpu.VMEM((NB, H, W, SEG, FOLD), jnp.float32),
            pltpu.VMEM((NB, SEG, FOLD), jnp.float32),
            pltpu.VMEM((NB, SEG, SEG, FOLD), jnp.float32),
        ],
    )(xt, xt,
      off_conv_w.reshape(3, FOLD), off_conv_b.reshape(1, 1),
      off_fc1_w, off_fc1_b.reshape(1, SEG),
      off_fc2_w, off_fc2_b.reshape(1, G),
      w_conv_w, w_conv_b.reshape(1, G))
    return jnp.transpose(out, (2, 3, 0, 1))


# speedup vs baseline: 1.1102x; 1.1102x over previous
import jax
import jax.numpy as jnp
from jax import lax
from jax.experimental import pallas as pl
from jax.experimental.pallas import tpu as pltpu

SEG = 8
NB = 8
F = NB * SEG
C = 512
FOLD = 128
H = 28
W = 28
G = 2
NCB = C // FOLD


def _sigmoid(v):
    return 1.0 / (1.0 + jnp.exp(-v))


def _body(fold_ref, pass_ref, ocw_ref, ocb_ref, f1w_ref, f1b_ref, f2w_ref,
          f2b_ref, wcw_ref, wcb_ref, out_ref, cache_ref, pool_ref, m_ref):
    p = pl.program_id(0)
    b = pl.program_id(1)

    @pl.when(p == 0)
    def _pool_phase():
        blk = fold_ref[...]
        cache_ref[b] = blk
        pool_ref[b] = jnp.sum(blk, axis=(0, 1)) * (1.0 / (H * W))
        out_ref[...] = pass_ref[...]

    @pl.when((p == 1) | (p == 2))
    def _pass_phase():
        out_ref[...] = pass_ref[...]

    @pl.when(p == 3)
    def _blend_phase():
        @pl.when(b == 0)
        def _nets():
            pool = pool_ref[...]
            zero = jnp.zeros((NB, 1, FOLD), jnp.float32)
            p_m1 = jnp.concatenate([zero, pool[:, : SEG - 1]], axis=1)
            p_p1 = jnp.concatenate([pool[:, 1:], zero], axis=1)

            ocw = ocw_ref[...]
            t0 = (jnp.sum(p_m1 * ocw[0][None, None, :], axis=-1)
                  + jnp.sum(pool * ocw[1][None, None, :], axis=-1)
                  + jnp.sum(p_p1 * ocw[2][None, None, :], axis=-1)
                  + ocb_ref[0, 0])
            t1 = jnp.maximum(
                jnp.dot(t0, f1w_ref[...], preferred_element_type=jnp.float32)
                + f1b_ref[0][None, :], 0.0)
            t2 = (jnp.dot(t1, f2w_ref[...], preferred_element_type=jnp.float32)
                  + f2b_ref[0][None, :])
            x_offset = 4.0 * (_sigmoid(t2) - 0.5)
            off_bs = jnp.tile(x_offset, (1, SEG // G))

            off0f = jnp.floor(off_bs)
            o0 = jnp.clip(off0f.astype(jnp.int32), 0, SEG - 1)
            o1 = jnp.clip(off0f.astype(jnp.int32) + 1, 0, SEG - 1)
            w1 = off_bs - off0f
            w0 = 1.0 - w1

            wcw = wcw_ref[...]
            xw = []
            for g in range(G):
                ug = (jnp.sum(p_m1 * wcw[0, :, g][None, None, :], axis=-1)
                      + jnp.sum(pool * wcw[1, :, g][None, None, :], axis=-1)
                      + jnp.sum(p_p1 * wcw[2, :, g][None, None, :], axis=-1)
                      + wcb_ref[0, g])
                xw.append(2.0 * _sigmoid(ug))

            fi = lax.broadcasted_iota(jnp.int32, (NB, SEG, SEG), 2)
            m4 = (jnp.where(fi == o0[:, :, None], w0[:, :, None], 0.0)
                  + jnp.where(fi == o1[:, :, None], w1[:, :, None], 0.0))
            xw_chan = jnp.concatenate(
                [jnp.broadcast_to(xw[0][:, :, None], (NB, SEG, FOLD // G)),
                 jnp.broadcast_to(xw[1][:, :, None], (NB, SEG, FOLD // G))],
                axis=-1)
            m_ref[...] = m4[:, :, :, None] * xw_chan[:, :, None, :]

        m = m_ref[b]
        blk = cache_ref[b]
        acc = blk[:, :, 0, :][:, :, None, :] * m[:, 0, :][None, None, :, :]
        for f in range(1, SEG):
            acc += blk[:, :, f, :][:, :, None, :] * m[:, f, :][None, None, :, :]
        out_ref[...] = acc


def kernel(x, off_conv_w, off_conv_b, off_fc1_w, off_fc1_b, off_fc2_w,
           off_fc2_b, w_conv_w, w_conv_b):
    xt = jnp.transpose(x, (2, 3, 0, 1))
    blk = (H, W, SEG, FOLD)
    out = pl.pallas_call(
        _body,
        grid=(NCB, NB),
        in_specs=[
            pl.BlockSpec(blk, lambda p, b: (0, 0,
                                            jnp.where(p == 0, b, NB - 1), 0)),
            pl.BlockSpec(blk, lambda p, b: (0, 0,
                                            jnp.where(p == 3, NB - 1, b),
                                            jnp.where(p == 3, 3, p + 1))),
            pl.BlockSpec((3, FOLD), lambda p, b: (0, 0)),
            pl.BlockSpec((1, 1), lambda p, b: (0, 0)),
            pl.BlockSpec((SEG, SEG), lambda p, b: (0, 0)),
            pl.BlockSpec((1, SEG), lambda p, b: (0, 0)),
            pl.BlockSpec((SEG, G), lambda p, b: (0, 0)),
            pl.BlockSpec((1, G), lambda p, b: (0, 0)),
            pl.BlockSpec((3, FOLD, G), lambda p, b: (0, 0, 0)),
            pl.BlockSpec((1, G), lambda p, b: (0, 0)),
        ],
        out_specs=pl.BlockSpec(blk, lambda p, b: (0, 0, b,
                                                  jnp.where(p == 3, 0, p + 1))),
        out_shape=jax.ShapeDtypeStruct((H, W, F, C), jnp.float32),
        scratch_shapes=[
            pltpu.VMEM((NB, H, W, SEG, FOLD), jnp.float32),
            pltpu.VMEM((NB, SEG, FOLD), jnp.float32),
            pltpu.VMEM((NB, SEG, SEG, FOLD), jnp.float32),
        ],
    )(xt, xt,
      off_conv_w.reshape(3, FOLD), off_conv_b.reshape(1, 1),
      off_fc1_w, off_fc1_b.reshape(1, SEG),
      off_fc2_w, off_fc2_b.reshape(1, G),
      w_conv_w, w_conv_b.reshape(1, G))
    return jnp.transpose(out, (2, 3, 0, 1))
